# Initial kernel scaffold; baseline (speedup 1.0000x reference)
import functools

import jax
import jax.numpy as jnp
from jax import lax
from jax.experimental import pallas as pl
from jax.experimental.pallas import tpu as pltpu
from jax.experimental.pallas import tpu_sc as plsc

CHUNK = 128
NCORES = 2
NSUB = 16
NTILES = NCORES * NSUB


def _mesh():
    return plsc.VectorSubcoreMesh(core_axis_name="c", subcore_axis_name="s")


def _deg_kernel(n_pad, nch, dw):
    rpt = n_pad // NSUB

    @functools.partial(
        pl.kernel,
        out_type=jax.ShapeDtypeStruct((NCORES, n_pad, dw), jnp.float32),
        mesh=_mesh(),
        scratch_types=[
            pltpu.VMEM((nch, CHUNK), jnp.int32),
            pltpu.VMEM((CHUNK, dw), jnp.float32),
            pltpu.VMEM_SHARED((n_pad, dw), jnp.float32),
        ],
    )
    def k(dst_hbm, zeros_hbm, ones_hbm, out_hbm, dst_v, ones_v, acc):
        c = lax.axis_index("c")
        s = lax.axis_index("s")
        wid = s * NCORES + c
        pltpu.sync_copy(dst_hbm.at[pl.ds(wid * nch, nch)], dst_v)
        pltpu.sync_copy(ones_hbm, ones_v)
        pltpu.sync_copy(zeros_hbm.at[pl.ds(s * rpt, rpt)], acc.at[pl.ds(s * rpt, rpt)])
        plsc.subcore_barrier()

        def body(j, carry):
            pltpu.sync_copy(ones_v, acc.at[dst_v.at[j]], add=True)
            return carry

        lax.fori_loop(0, nch, body, 0)
        plsc.subcore_barrier()
        pltpu.sync_copy(acc.at[pl.ds(s * rpt, rpt)], out_hbm.at[c, pl.ds(s * rpt, rpt)])

    return k


def _scatter_kernel(n_pad, nch, f):
    rpt = n_pad // NSUB

    @functools.partial(
        pl.kernel,
        out_type=jax.ShapeDtypeStruct((NCORES, n_pad, f), jnp.float32),
        mesh=_mesh(),
        scratch_types=[
            pltpu.VMEM((nch, CHUNK), jnp.int32),
            pltpu.VMEM((nch, CHUNK), jnp.int32),
            pltpu.VMEM((CHUNK, f), jnp.float32),
            pltpu.VMEM((CHUNK, f), jnp.float32),
            pltpu.VMEM_SHARED((n_pad, f), jnp.float32),
            pltpu.SemaphoreType.DMA,
            pltpu.SemaphoreType.DMA,
        ],
    )
    def k(src_hbm, dst_hbm, y_hbm, zeros_hbm, out_hbm,
          src_v, dst_v, r0, r1, acc, sem0, sem1):
        c = lax.axis_index("c")
        s = lax.axis_index("s")
        wid = s * NCORES + c
        pltpu.sync_copy(src_hbm.at[pl.ds(wid * nch, nch)], src_v)
        pltpu.sync_copy(dst_hbm.at[pl.ds(wid * nch, nch)], dst_v)
        pltpu.sync_copy(zeros_hbm.at[pl.ds(s * rpt, rpt)], acc.at[pl.ds(s * rpt, rpt)])
        plsc.subcore_barrier()

        pltpu.async_copy(y_hbm.at[src_v.at[0]], r0, sem0)

        def body(i, carry):
            j0 = 2 * i
            pltpu.async_copy(y_hbm.at[src_v.at[j0 + 1]], r1, sem1)
            pltpu.make_async_copy(y_hbm.at[src_v.at[j0]], r0, sem0).wait()
            pltpu.sync_copy(r0, acc.at[dst_v.at[j0]], add=True)
            pltpu.async_copy(y_hbm.at[src_v.at[j0 + 2]], r0, sem0)
            pltpu.make_async_copy(y_hbm.at[src_v.at[j0 + 1]], r1, sem1).wait()
            pltpu.sync_copy(r1, acc.at[dst_v.at[j0 + 1]], add=True)
            return carry

        lax.fori_loop(0, nch // 2 - 1, body, 0)
        j0 = nch - 2
        pltpu.async_copy(y_hbm.at[src_v.at[j0 + 1]], r1, sem1)
        pltpu.make_async_copy(y_hbm.at[src_v.at[j0]], r0, sem0).wait()
        pltpu.sync_copy(r0, acc.at[dst_v.at[j0]], add=True)
        pltpu.make_async_copy(y_hbm.at[src_v.at[j0 + 1]], r1, sem1).wait()
        pltpu.sync_copy(r1, acc.at[dst_v.at[j0 + 1]], add=True)

        plsc.subcore_barrier()
        pltpu.sync_copy(acc.at[pl.ds(s * rpt, rpt)], out_hbm.at[c, pl.ds(s * rpt, rpt)])

    return k


def _tc_prep(deg2, x, w1, n):
    h = w1.shape[1]

    def body(deg_ref, x_ref, w_ref, y_ref, dinv_ref):
        d = deg_ref[0, :n, 0:1] + deg_ref[1, :n, 0:1] + 1.0
        dinv = lax.rsqrt(d)
        xt = jnp.dot(x_ref[...], w_ref[...], preferred_element_type=jnp.float32)
        y_ref[...] = xt * dinv
        dinv_ref[...] = dinv

    return pl.pallas_call(
        body,
        out_shape=[
            jax.ShapeDtypeStruct((n, h), jnp.float32),
            jax.ShapeDtypeStruct((n, 1), jnp.float32),
        ],
    )(deg2, x, w1)


def _tc_mid(s1, y1, dinv, b1, w2p, n):
    w = w2p.shape[1]

    def body(s_ref, y_ref, dinv_ref, b_ref, w_ref, o_ref):
        o1 = (s_ref[0, :n, :] + s_ref[1, :n, :] + y_ref[...]) * dinv_ref[...] + b_ref[...]
        hid = jnp.maximum(o1, 0.0)
        o_ref[...] = jnp.dot(hid, w_ref[...], preferred_element_type=jnp.float32) * dinv_ref[...]

    return pl.pallas_call(
        body,
        out_shape=jax.ShapeDtypeStruct((n, w), jnp.float32),
    )(s1, y1, dinv, b1, w2p)


def _tc_final(s2, y2, dinv, b2, n, c_out):
    def body(s_ref, y_ref, dinv_ref, b_ref, o_ref):
        o = (s_ref[0, :n, :] + s_ref[1, :n, :] + y_ref[...]) * dinv_ref[...]
        logits = o[:, 0:c_out] + b_ref[...]
        m = jnp.max(logits, axis=1, keepdims=True)
        ex = jnp.exp(logits - m)
        lse = jnp.log(jnp.sum(ex, axis=1, keepdims=True))
        o_ref[...] = logits - m - lse

    return pl.pallas_call(
        body,
        out_shape=jax.ShapeDtypeStruct((n, c_out), jnp.float32),
    )(s2, y2, dinv, b2)


def kernel(x, edge_index, W1, b1, W2, b2):
    n, _ = x.shape
    h = W1.shape[1]
    c_out = W2.shape[1]
    e = edge_index.shape[1]

    nch = -(-e // (NTILES * CHUNK))
    nch += nch % 2
    e_pad = NTILES * CHUNK * nch
    n_pad = ((n + 1 + NSUB - 1) // NSUB) * NSUB
    dw = 16
    w2w = 16

    src = jnp.concatenate(
        [edge_index[0], jnp.zeros((e_pad - e,), jnp.int32)]).reshape(-1, CHUNK)
    dst = jnp.concatenate(
        [edge_index[1], jnp.full((e_pad - e,), n, jnp.int32)]).reshape(-1, CHUNK)

    zeros_dw = jnp.zeros((n_pad, dw), jnp.float32)
    ones_dw = jnp.ones((CHUNK, dw), jnp.float32)
    zeros_h = jnp.zeros((n_pad, h), jnp.float32)
    zeros_w2 = jnp.zeros((n_pad, w2w), jnp.float32)
    w2p = jnp.pad(W2, ((0, 0), (0, w2w - c_out)))

    deg2 = _deg_kernel(n_pad, nch, dw)(dst, zeros_dw, ones_dw)
    y1, dinv = _tc_prep(deg2, x, W1, n)
    s1 = _scatter_kernel(n_pad, nch, h)(src, dst, y1, zeros_h)
    y2 = _tc_mid(s1, y1, dinv, b1.reshape(1, h), w2p, n)
    s2 = _scatter_kernel(n_pad, nch, w2w)(src, dst, y2, zeros_w2)
    return _tc_final(s2, y2, dinv, b2.reshape(1, c_out), n, c_out)


# speedup vs baseline: 32.5142x; 32.5142x over previous
import functools

import jax
import jax.numpy as jnp
from jax import lax
from jax.experimental import pallas as pl
from jax.experimental.pallas import tpu as pltpu
from jax.experimental.pallas import tpu_sc as plsc

CHUNK = 128
NCORES = 2
NSUB = 16
NTILES = NCORES * NSUB


def _mesh():
    return plsc.VectorSubcoreMesh(core_axis_name="c", subcore_axis_name="s")


def _deg_kernel(n_pad, nch, dw):
    rpt = n_pad // NSUB

    @functools.partial(
        pl.kernel,
        out_type=jax.ShapeDtypeStruct((NCORES, n_pad, dw), jnp.float32),
        mesh=_mesh(),
        compiler_params=pltpu.CompilerParams(use_tc_tiling_on_sc=False),
        scratch_types=[
            pltpu.VMEM((nch, CHUNK), jnp.int32),
            pltpu.VMEM((CHUNK, dw), jnp.float32),
            pltpu.VMEM_SHARED((n_pad, dw), jnp.float32),
        ],
    )
    def k(dst_hbm, zeros_hbm, ones_hbm, out_hbm, dst_v, ones_v, acc):
        c = lax.axis_index("c")
        s = lax.axis_index("s")
        wid = s * NCORES + c
        pltpu.sync_copy(dst_hbm.at[pl.ds(wid * nch, nch)], dst_v)
        pltpu.sync_copy(ones_hbm, ones_v)
        pltpu.sync_copy(zeros_hbm.at[pl.ds(s * rpt, rpt)], acc.at[pl.ds(s * rpt, rpt)])
        plsc.subcore_barrier()

        def body(j, carry):
            pltpu.sync_copy(ones_v, acc.at[dst_v.at[j]], add=True)
            return carry

        lax.fori_loop(0, nch, body, 0)
        plsc.subcore_barrier()
        pltpu.sync_copy(acc.at[pl.ds(s * rpt, rpt)], out_hbm.at[c, pl.ds(s * rpt, rpt)])

    return k


def _scatter_kernel(n_pad, nch, f):
    rpt = n_pad // NSUB

    @functools.partial(
        pl.kernel,
        out_type=jax.ShapeDtypeStruct((NCORES, n_pad, f), jnp.float32),
        mesh=_mesh(),
        compiler_params=pltpu.CompilerParams(use_tc_tiling_on_sc=False),
        scratch_types=[
            pltpu.VMEM((nch, CHUNK), jnp.int32),
            pltpu.VMEM((nch, CHUNK), jnp.int32),
            pltpu.VMEM((CHUNK, f), jnp.float32),
            pltpu.VMEM((CHUNK, f), jnp.float32),
            pltpu.VMEM_SHARED((n_pad, f), jnp.float32),
            pltpu.SemaphoreType.DMA,
            pltpu.SemaphoreType.DMA,
        ],
    )
    def k(src_hbm, dst_hbm, y_hbm, zeros_hbm, out_hbm,
          src_v, dst_v, r0, r1, acc, sem0, sem1):
        c = lax.axis_index("c")
        s = lax.axis_index("s")
        wid = s * NCORES + c
        pltpu.sync_copy(src_hbm.at[pl.ds(wid * nch, nch)], src_v)
        pltpu.sync_copy(dst_hbm.at[pl.ds(wid * nch, nch)], dst_v)
        pltpu.sync_copy(zeros_hbm.at[pl.ds(s * rpt, rpt)], acc.at[pl.ds(s * rpt, rpt)])
        plsc.subcore_barrier()

        pltpu.async_copy(y_hbm.at[src_v.at[0]], r0, sem0)

        def body(i, carry):
            j0 = 2 * i
            pltpu.async_copy(y_hbm.at[src_v.at[j0 + 1]], r1, sem1)
            pltpu.make_async_copy(y_hbm.at[src_v.at[j0]], r0, sem0).wait()
            pltpu.sync_copy(r0, acc.at[dst_v.at[j0]], add=True)
            pltpu.async_copy(y_hbm.at[src_v.at[j0 + 2]], r0, sem0)
            pltpu.make_async_copy(y_hbm.at[src_v.at[j0 + 1]], r1, sem1).wait()
            pltpu.sync_copy(r1, acc.at[dst_v.at[j0 + 1]], add=True)
            return carry

        lax.fori_loop(0, nch // 2 - 1, body, 0)
        j0 = nch - 2
        pltpu.async_copy(y_hbm.at[src_v.at[j0 + 1]], r1, sem1)
        pltpu.make_async_copy(y_hbm.at[src_v.at[j0]], r0, sem0).wait()
        pltpu.sync_copy(r0, acc.at[dst_v.at[j0]], add=True)
        pltpu.make_async_copy(y_hbm.at[src_v.at[j0 + 1]], r1, sem1).wait()
        pltpu.sync_copy(r1, acc.at[dst_v.at[j0 + 1]], add=True)

        plsc.subcore_barrier()
        pltpu.sync_copy(acc.at[pl.ds(s * rpt, rpt)], out_hbm.at[c, pl.ds(s * rpt, rpt)])

    return k


def _tc_prep(deg2, x, w1, n):
    h = w1.shape[1]

    def body(deg_ref, x_ref, w_ref, y_ref, dinv_ref):
        d = deg_ref[0, :n, 0:1] + deg_ref[1, :n, 0:1] + 1.0
        dinv = lax.rsqrt(d)
        xt = jnp.dot(x_ref[...], w_ref[...], preferred_element_type=jnp.float32)
        y_ref[...] = xt * dinv
        dinv_ref[...] = dinv

    return pl.pallas_call(
        body,
        out_shape=[
            jax.ShapeDtypeStruct((n, h), jnp.float32),
            jax.ShapeDtypeStruct((n, 1), jnp.float32),
        ],
    )(deg2, x, w1)


def _tc_mid(s1, y1, dinv, b1, w2p, n):
    w = w2p.shape[1]

    def body(s_ref, y_ref, dinv_ref, b_ref, w_ref, o_ref):
        o1 = (s_ref[0, :n, :] + s_ref[1, :n, :] + y_ref[...]) * dinv_ref[...] + b_ref[...]
        hid = jnp.maximum(o1, 0.0)
        o_ref[...] = jnp.dot(hid, w_ref[...], preferred_element_type=jnp.float32) * dinv_ref[...]

    return pl.pallas_call(
        body,
        out_shape=jax.ShapeDtypeStruct((n, w), jnp.float32),
    )(s1, y1, dinv, b1, w2p)


def _tc_final(s2, y2, dinv, b2, n, c_out):
    def body(s_ref, y_ref, dinv_ref, b_ref, o_ref):
        o = (s_ref[0, :n, :] + s_ref[1, :n, :] + y_ref[...]) * dinv_ref[...]
        logits = o[:, 0:c_out] + b_ref[...]
        m = jnp.max(logits, axis=1, keepdims=True)
        ex = jnp.exp(logits - m)
        lse = jnp.log(jnp.sum(ex, axis=1, keepdims=True))
        o_ref[...] = logits - m - lse

    return pl.pallas_call(
        body,
        out_shape=jax.ShapeDtypeStruct((n, c_out), jnp.float32),
    )(s2, y2, dinv, b2)


def kernel(x, edge_index, W1, b1, W2, b2):
    n, _ = x.shape
    h = W1.shape[1]
    c_out = W2.shape[1]
    e = edge_index.shape[1]

    nch = -(-e // (NTILES * CHUNK))
    nch += nch % 2
    e_pad = NTILES * CHUNK * nch
    n_pad = -(-(n + 1) // (NSUB * 8)) * (NSUB * 8)
    dw = 16
    w2w = 16

    src = jnp.concatenate(
        [edge_index[0], jnp.zeros((e_pad - e,), jnp.int32)]).reshape(-1, CHUNK)
    dst = jnp.concatenate(
        [edge_index[1], jnp.full((e_pad - e,), n, jnp.int32)]).reshape(-1, CHUNK)

    zeros_dw = jnp.zeros((n_pad, dw), jnp.float32)
    ones_dw = jnp.ones((CHUNK, dw), jnp.float32)
    zeros_h = jnp.zeros((n_pad, h), jnp.float32)
    zeros_w2 = jnp.zeros((n_pad, w2w), jnp.float32)
    w2p = jnp.pad(W2, ((0, 0), (0, w2w - c_out)))

    deg2 = _deg_kernel(n_pad, nch, dw)(dst, zeros_dw, ones_dw)
    y1, dinv = _tc_prep(deg2, x, W1, n)
    s1 = _scatter_kernel(n_pad, nch, h)(src, dst, y1, zeros_h)
    y2 = _tc_mid(s1, y1, dinv, b1.reshape(1, h), w2p, n)
    s2 = _scatter_kernel(n_pad, nch, w2w)(src, dst, y2, zeros_w2)
    return _tc_final(s2, y2, dinv, b2.reshape(1, c_out), n, c_out)
